# baseline (speedup 1.0000x reference)
import functools

import jax
import jax.numpy as jnp
from jax import lax
from jax.experimental import pallas as pl
from jax.experimental.pallas import tpu as pltpu
from jax.experimental.pallas import tpu_sc as plsc

P, C, H, W, B = 96000, 64, 400, 400, 8
N = B * H * W
NW = 32
RPW = N // NW
ZPAD = 1024
TPAD = 1024
GROUP = 1280
NG = P // GROUP
VPG = GROUP // 16
PG = 400
NPG = RPW // PG


def _sc_body(feat, bq, xq, yq, pidrows, table, mapm, mbuf, stage_b, stage_x,
             stage_y, sidx, pvr, gbuf2, accv, didx16, dval16r, g16r, mp2,
             gidx, prow, gidx2, prow2, semz, semg, sems):
    sax = lax.axis_index("s")
    cax = lax.axis_index("c")
    wid = sax * 2 + cax
    base = wid * RPW
    tbase = N + wid * TPAD
    lane = lax.iota(jnp.int32, 16)

    for r in range(250):
        mbuf[r, pl.ds(0, 16)] = jnp.full((16,), -1, jnp.int32)

    def _minit(i, carry):
        ds_ = [pltpu.async_copy(
            mbuf, mapm.at[pl.ds(base + (i * 8 + u) * 250, 250)], semz)
            for u in range(8)]
        for dd in ds_:
            dd.wait()
        return carry
    lax.fori_loop(0, RPW // 250 // 8, _minit, 0)

    def _stage(g):
        pltpu.sync_copy(bq.at[pl.ds(g * GROUP, GROUP)], stage_b)
        pltpu.sync_copy(xq.at[pl.ds(g * GROUP, GROUP)], stage_x)
        pltpu.sync_copy(yq.at[pl.ds(g * GROUP, GROUP)], stage_y)

    def _build(g):
        def body(k, carry):
            k16 = k * 16
            bb = stage_b[pl.ds(k16, 16)]
            xx = stage_x[pl.ds(k16, 16)]
            yy = stage_y[pl.ds(k16, 16)]
            o = bb * (H * W) + xx * W + yy
            lo = o - base
            inm = (lo >= 0) & (lo < RPW)
            pid = g * GROUP + k16 + lane
            sidx[pl.ds(k16, 16)] = jnp.where(inm, o, tbase + (pid & (TPAD - 1)))
            return carry
        lax.fori_loop(0, VPG, body, 0)

    def _p1(g, carry):
        _stage(g)
        _build(g)
        pltpu.sync_copy(pidrows.at[pl.ds(g * GROUP, GROUP)], pvr)
        pltpu.async_copy(pvr, mapm.at[sidx], sems).wait()
        return carry
    lax.fori_loop(0, NG, _p1, 0)

    def _p2(g, carry):
        _stage(g)
        _build(g)
        pltpu.async_copy(mapm.at[sidx], gbuf2, semg).wait()
        accv[...] = jnp.zeros((16,), jnp.int32)

        def det(k, carry2):
            k16 = k * 16
            ov = sidx[pl.ds(k16, 16)]
            pid = g * GROUP + k16 + lane
            gv = gbuf2[k16, pl.ds(0, 16)]
            for l in range(1, 16):
                gv = jnp.where(lane == l, gbuf2[k16 + l, pl.ds(0, 16)], gv)
            own = ov < N
            bad = own & (gv < pid)
            accv[...] = accv[...] | jnp.where(bad, jnp.int32(1), jnp.int32(0))
            return carry2
        lax.fori_loop(0, VPG, det, 0)
        av = accv[...]
        t = (av[0] + av[1] + av[2] + av[3] + av[4] + av[5] + av[6] + av[7]
             + av[8] + av[9] + av[10] + av[11] + av[12] + av[13] + av[14]
             + av[15])

        def scan(k, carry2):
            k16 = k * 16
            ov = sidx[pl.ds(k16, 16)]
            for l in range(16):
                ol = ov[l]
                gl = gbuf2[k16 + l, pl.ds(0, 16)][0]
                pid_l = g * GROUP + k16 + l
                need = jnp.where((ol < N) & (gl < pid_l),
                                 jnp.int32(1), jnp.int32(0))

                def fix(r, carry3):
                    didx16[...] = jnp.where(
                        lane == 0, ol, tbase + ((pid_l + lane) & (TPAD - 1)))
                    pltpu.async_copy(mapm.at[didx16], g16r, semg).wait()
                    cur = g16r[0, pl.ds(0, 16)][0]
                    wr = jnp.where(pid_l > cur, jnp.int32(1), jnp.int32(0))

                    def put(r2, carry4):
                        for rr in range(16):
                            dval16r[rr, pl.ds(0, 16)] = (
                                jnp.full((16,), 1, jnp.int32) * pid_l)
                        pltpu.async_copy(dval16r, mapm.at[didx16], sems).wait()
                        return carry4
                    lax.fori_loop(0, wr, put, 0)
                    return carry3
                lax.fori_loop(0, need, fix, 0)
            return carry2
        lax.fori_loop(0, VPG * jnp.minimum(t, 1), scan, 0)
        return carry
    lax.fori_loop(0, NG, _p2, 0)

    def _p3(q, carry):
        row0 = base + q * 2 * PG

        def _mk(gi, roff):
            for k in range(PG // 16):
                k16 = k * 16
                m = mp2[roff + k16, pl.ds(0, 16)]
                for l in range(1, 16):
                    m = jnp.where(
                        lane == l, mp2[roff + k16 + l, pl.ds(0, 16)], m)
                spread = P + ((row0 + roff + k16 + lane) & (ZPAD - 1))
                gi[pl.ds(k16, 16)] = jnp.where(m < 0, spread, m)

        pltpu.sync_copy(mapm.at[pl.ds(row0, 2 * PG)], mp2)
        _mk(gidx, 0)
        dg1 = pltpu.async_copy(feat.at[gidx], prow, semg)
        _mk(gidx2, PG)
        dg2 = pltpu.async_copy(feat.at[gidx2], prow2, semg)
        dg1.wait()
        dw1 = pltpu.async_copy(prow, table.at[pl.ds(row0, PG)], sems)
        dg2.wait()
        dw2 = pltpu.async_copy(prow2, table.at[pl.ds(row0 + PG, PG)], sems)
        dw1.wait()
        dw2.wait()
        return carry
    lax.fori_loop(0, NPG // 2, _p3, 0)


_sc_scatter = functools.partial(
    pl.kernel,
    out_type=(jax.ShapeDtypeStruct((N, C), jnp.float32),
              jax.ShapeDtypeStruct((N + NW * TPAD + 64, 16), jnp.int32)),
    mesh=plsc.VectorSubcoreMesh(core_axis_name="c", subcore_axis_name="s"),
    compiler_params=pltpu.CompilerParams(use_tc_tiling_on_sc=False),
    scratch_types=[
        pltpu.VMEM((250, 16), jnp.int32),
        pltpu.VMEM((GROUP,), jnp.int32),
        pltpu.VMEM((GROUP,), jnp.int32),
        pltpu.VMEM((GROUP,), jnp.int32),
        pltpu.VMEM((GROUP,), jnp.int32),
        pltpu.VMEM((GROUP, 16), jnp.int32),
        pltpu.VMEM((GROUP, 16), jnp.int32),
        pltpu.VMEM((16,), jnp.int32),
        pltpu.VMEM((16,), jnp.int32),
        pltpu.VMEM((16, 16), jnp.int32),
        pltpu.VMEM((16, 16), jnp.int32),
        pltpu.VMEM((2 * PG, 16), jnp.int32),
        pltpu.VMEM((PG,), jnp.int32),
        pltpu.VMEM((PG, C), jnp.float32),
        pltpu.VMEM((PG,), jnp.int32),
        pltpu.VMEM((PG, C), jnp.float32),
        pltpu.SemaphoreType.DMA,
        pltpu.SemaphoreType.DMA,
        pltpu.SemaphoreType.DMA,
    ],
)(_sc_body)


def _t_body(in_ref, out_ref):
    out_ref[...] = jnp.transpose(in_ref[...], (0, 3, 1, 2))


XB = 16
_transpose = pl.pallas_call(
    _t_body,
    grid=(B, H // XB),
    in_specs=[pl.BlockSpec((1, XB, W, C), lambda ib, ix: (ib, ix, 0, 0))],
    out_specs=pl.BlockSpec((1, C, XB, W), lambda ib, ix: (ib, 0, ix, 0)),
    out_shape=jax.ShapeDtypeStruct((B, C, H, W), jnp.float32),
)


def kernel(pillar_features, coors, batch_size):
    ci = coors.astype(jnp.int32)
    bq = ci[:, 0]
    xq = ci[:, 1]
    yq = ci[:, 2]
    feat_ext = jnp.concatenate(
        [pillar_features.astype(jnp.float32),
         jnp.zeros((ZPAD, C), jnp.float32)], axis=0)
    pidrows = jnp.broadcast_to(
        jnp.arange(P, dtype=jnp.int32)[:, None], (P, 16))
    table, _ = _sc_scatter(feat_ext, bq, xq, yq, pidrows)
    return _transpose(table.reshape(B, H, W, C))


# speedup vs baseline: 1.0830x; 1.0830x over previous
import functools

import jax
import jax.numpy as jnp
from jax import lax
from jax.experimental import pallas as pl
from jax.experimental.pallas import tpu as pltpu
from jax.experimental.pallas import tpu_sc as plsc

P, C, H, W, B = 96000, 64, 400, 400, 8
N = B * H * W
NW = 32
RPW = N // NW
ZPAD = 1024
TPAD = 1024
GROUP = 2400
NG = P // GROUP
VPG = GROUP // 16
PG = 400
NPG = RPW // PG


def _sc_body(feat, bq, xq, yq, pidrows, table, mapm, mbuf, stage_b, stage_x,
             stage_y, sidx, pvr, gbuf2, accv, didx16, dval16r, g16r, mp2,
             gidx, prow, semz, semg, sems):
    sax = lax.axis_index("s")
    cax = lax.axis_index("c")
    wid = sax * 2 + cax
    base = wid * RPW
    tbase = N + wid * TPAD
    lane = lax.iota(jnp.int32, 16)

    for r in range(250):
        mbuf[r, pl.ds(0, 16)] = jnp.full((16,), -1, jnp.int32)

    def _minit(i, carry):
        pltpu.sync_copy(mbuf, mapm.at[pl.ds(base + i * 250, 250)])
        return carry
    lax.fori_loop(0, RPW // 250, _minit, 0)

    def _stage(g, refs):
        ds_ = [pltpu.async_copy(src_ref.at[pl.ds(g * GROUP, GROUP)], dst, semz)
               for src_ref, dst in refs]
        for dd in ds_:
            dd.wait()

    def _build(g):
        def body(k, carry):
            k16 = k * 16
            bb = stage_b[pl.ds(k16, 16)]
            xx = stage_x[pl.ds(k16, 16)]
            yy = stage_y[pl.ds(k16, 16)]
            o = bb * (H * W) + xx * W + yy
            lo = o - base
            inm = (lo >= 0) & (lo < RPW)
            pid = g * GROUP + k16 + lane
            sidx[pl.ds(k16, 16)] = jnp.where(inm, o, tbase + (pid & (TPAD - 1)))
            return carry
        lax.fori_loop(0, VPG, body, 0)

    def _p1(g, carry):
        _stage(g, [(bq, stage_b), (xq, stage_x), (yq, stage_y),
                   (pidrows, pvr)])
        _build(g)
        pltpu.async_copy(pvr, mapm.at[sidx], sems).wait()
        return carry
    lax.fori_loop(0, NG, _p1, 0)

    def _p2(g, carry):
        _stage(g, [(bq, stage_b), (xq, stage_x), (yq, stage_y)])
        _build(g)
        pltpu.async_copy(mapm.at[sidx], gbuf2, semg).wait()
        accv[...] = jnp.zeros((16,), jnp.int32)

        def det(k, carry2):
            k16 = k * 16
            ov = sidx[pl.ds(k16, 16)]
            pid = g * GROUP + k16 + lane
            gv = gbuf2[k16, pl.ds(0, 16)]
            for l in range(1, 16):
                gv = jnp.where(lane == l, gbuf2[k16 + l, pl.ds(0, 16)], gv)
            own = ov < N
            bad = own & (gv < pid)
            accv[...] = accv[...] | jnp.where(bad, jnp.int32(1), jnp.int32(0))
            return carry2
        lax.fori_loop(0, VPG, det, 0)
        av = accv[...]
        t = (av[0] + av[1] + av[2] + av[3] + av[4] + av[5] + av[6] + av[7]
             + av[8] + av[9] + av[10] + av[11] + av[12] + av[13] + av[14]
             + av[15])

        def scan(k, carry2):
            k16 = k * 16
            ov = sidx[pl.ds(k16, 16)]
            for l in range(16):
                ol = ov[l]
                gl = gbuf2[k16 + l, pl.ds(0, 16)][0]
                pid_l = g * GROUP + k16 + l
                need = jnp.where((ol < N) & (gl < pid_l),
                                 jnp.int32(1), jnp.int32(0))

                def fix(r, carry3):
                    didx16[...] = jnp.where(
                        lane == 0, ol, tbase + ((pid_l + lane) & (TPAD - 1)))
                    pltpu.async_copy(mapm.at[didx16], g16r, semg).wait()
                    cur = g16r[0, pl.ds(0, 16)][0]
                    wr = jnp.where(pid_l > cur, jnp.int32(1), jnp.int32(0))

                    def put(r2, carry4):
                        for rr in range(16):
                            dval16r[rr, pl.ds(0, 16)] = (
                                jnp.full((16,), 1, jnp.int32) * pid_l)
                        pltpu.async_copy(dval16r, mapm.at[didx16], sems).wait()
                        return carry4
                    lax.fori_loop(0, wr, put, 0)
                    return carry3
                lax.fori_loop(0, need, fix, 0)
            return carry2
        lax.fori_loop(0, VPG * jnp.minimum(t, 1), scan, 0)
        return carry
    lax.fori_loop(0, NG, _p2, 0)

    def _p3(p, carry):
        row0 = base + p * PG
        pltpu.sync_copy(mapm.at[pl.ds(row0, PG)], mp2)
        for k in range(PG // 16):
            k16 = k * 16
            m = mp2[k16, pl.ds(0, 16)]
            for l in range(1, 16):
                m = jnp.where(lane == l, mp2[k16 + l, pl.ds(0, 16)], m)
            spread = P + ((row0 + k16 + lane) & (ZPAD - 1))
            gidx[pl.ds(k16, 16)] = jnp.where(m < 0, spread, m)
        pltpu.async_copy(feat.at[gidx], prow, semg).wait()
        pltpu.sync_copy(prow, table.at[pl.ds(row0, PG)])
        return carry
    lax.fori_loop(0, NPG, _p3, 0)


_sc_scatter = functools.partial(
    pl.kernel,
    out_type=(jax.ShapeDtypeStruct((N, C), jnp.float32),
              jax.ShapeDtypeStruct((N + NW * TPAD + 64, 16), jnp.int32)),
    mesh=plsc.VectorSubcoreMesh(core_axis_name="c", subcore_axis_name="s"),
    compiler_params=pltpu.CompilerParams(use_tc_tiling_on_sc=False),
    scratch_types=[
        pltpu.VMEM((250, 16), jnp.int32),
        pltpu.VMEM((GROUP,), jnp.int32),
        pltpu.VMEM((GROUP,), jnp.int32),
        pltpu.VMEM((GROUP,), jnp.int32),
        pltpu.VMEM((GROUP,), jnp.int32),
        pltpu.VMEM((GROUP, 16), jnp.int32),
        pltpu.VMEM((GROUP, 16), jnp.int32),
        pltpu.VMEM((16,), jnp.int32),
        pltpu.VMEM((16,), jnp.int32),
        pltpu.VMEM((16, 16), jnp.int32),
        pltpu.VMEM((16, 16), jnp.int32),
        pltpu.VMEM((PG, 16), jnp.int32),
        pltpu.VMEM((PG,), jnp.int32),
        pltpu.VMEM((PG, C), jnp.float32),
        pltpu.SemaphoreType.DMA,
        pltpu.SemaphoreType.DMA,
        pltpu.SemaphoreType.DMA,
    ],
)(_sc_body)


def _t_body(in_ref, out_ref):
    out_ref[...] = jnp.transpose(in_ref[...], (0, 3, 1, 2))


XB = 16
_transpose = pl.pallas_call(
    _t_body,
    grid=(B, H // XB),
    in_specs=[pl.BlockSpec((1, XB, W, C), lambda ib, ix: (ib, ix, 0, 0))],
    out_specs=pl.BlockSpec((1, C, XB, W), lambda ib, ix: (ib, 0, ix, 0)),
    out_shape=jax.ShapeDtypeStruct((B, C, H, W), jnp.float32),
)


def kernel(pillar_features, coors, batch_size):
    ci = coors.astype(jnp.int32)
    bq = ci[:, 0]
    xq = ci[:, 1]
    yq = ci[:, 2]
    feat_ext = jnp.concatenate(
        [pillar_features.astype(jnp.float32),
         jnp.zeros((ZPAD, C), jnp.float32)], axis=0)
    pidrows = jnp.broadcast_to(
        jnp.arange(P, dtype=jnp.int32)[:, None], (P, 16))
    table, _ = _sc_scatter(feat_ext, bq, xq, yq, pidrows)
    return _transpose(table.reshape(B, H, W, C))


# speedup vs baseline: 1.0832x; 1.0002x over previous
import functools

import jax
import jax.numpy as jnp
from jax import lax
from jax.experimental import pallas as pl
from jax.experimental.pallas import tpu as pltpu
from jax.experimental.pallas import tpu_sc as plsc

P, C, H, W, B = 96000, 64, 400, 400, 8
N = B * H * W
NW = 32
RPW = N // NW
ZPAD = 1024
TPAD = 1024
GROUP = 2400
NG = P // GROUP
VPG = GROUP // 16
PG = 400
NPG = RPW // PG


def _sc_body(feat, bq, xq, yq, pidrows, table, mapm, mbuf, stage_b, stage_x,
             stage_y, sidx, pvr, gbuf2, accv, didx16, dval16r, g16r, mp2,
             gidx, prow, semz, semg, sems):
    sax = lax.axis_index("s")
    cax = lax.axis_index("c")
    wid = sax * 2 + cax
    base = wid * RPW
    tbase = N + wid * TPAD
    lane = lax.iota(jnp.int32, 16)

    for r in range(250):
        mbuf[r, pl.ds(0, 16)] = jnp.full((16,), -1, jnp.int32)

    def _minit(i, carry):
        ds_ = [pltpu.async_copy(
            mbuf, mapm.at[pl.ds(base + (i * 8 + u) * 250, 250)], semz)
            for u in range(8)]
        for dd in ds_:
            dd.wait()
        return carry
    lax.fori_loop(0, RPW // 250 // 8, _minit, 0)

    def _stage(g, refs):
        ds_ = [pltpu.async_copy(src_ref.at[pl.ds(g * GROUP, GROUP)], dst, semz)
               for src_ref, dst in refs]
        for dd in ds_:
            dd.wait()

    def _build(g):
        def body(k, carry):
            k16 = k * 16
            bb = stage_b[pl.ds(k16, 16)]
            xx = stage_x[pl.ds(k16, 16)]
            yy = stage_y[pl.ds(k16, 16)]
            o = bb * (H * W) + xx * W + yy
            lo = o - base
            inm = (lo >= 0) & (lo < RPW)
            pid = g * GROUP + k16 + lane
            sidx[pl.ds(k16, 16)] = jnp.where(inm, o, tbase + (pid & (TPAD - 1)))
            return carry
        lax.fori_loop(0, VPG, body, 0)

    def _p1(g, carry):
        _stage(g, [(bq, stage_b), (xq, stage_x), (yq, stage_y),
                   (pidrows, pvr)])
        _build(g)
        pltpu.async_copy(pvr, mapm.at[sidx], sems).wait()
        return carry
    lax.fori_loop(0, NG, _p1, 0)

    def _p2(g, carry):
        _stage(g, [(bq, stage_b), (xq, stage_x), (yq, stage_y)])
        _build(g)
        pltpu.async_copy(mapm.at[sidx], gbuf2, semg).wait()
        accv[...] = jnp.zeros((16,), jnp.int32)

        def det(k, carry2):
            k16 = k * 16
            ov = sidx[pl.ds(k16, 16)]
            pid = g * GROUP + k16 + lane
            gv = gbuf2[k16, pl.ds(0, 16)]
            for l in range(1, 16):
                gv = jnp.where(lane == l, gbuf2[k16 + l, pl.ds(0, 16)], gv)
            own = ov < N
            bad = own & (gv < pid)
            accv[...] = accv[...] | jnp.where(bad, jnp.int32(1), jnp.int32(0))
            return carry2
        lax.fori_loop(0, VPG, det, 0)
        av = accv[...]
        t = (av[0] + av[1] + av[2] + av[3] + av[4] + av[5] + av[6] + av[7]
             + av[8] + av[9] + av[10] + av[11] + av[12] + av[13] + av[14]
             + av[15])

        def scan(k, carry2):
            k16 = k * 16
            ov = sidx[pl.ds(k16, 16)]
            for l in range(16):
                ol = ov[l]
                gl = gbuf2[k16 + l, pl.ds(0, 16)][0]
                pid_l = g * GROUP + k16 + l
                need = jnp.where((ol < N) & (gl < pid_l),
                                 jnp.int32(1), jnp.int32(0))

                def fix(r, carry3):
                    didx16[...] = jnp.where(
                        lane == 0, ol, tbase + ((pid_l + lane) & (TPAD - 1)))
                    pltpu.async_copy(mapm.at[didx16], g16r, semg).wait()
                    cur = g16r[0, pl.ds(0, 16)][0]
                    wr = jnp.where(pid_l > cur, jnp.int32(1), jnp.int32(0))

                    def put(r2, carry4):
                        for rr in range(16):
                            dval16r[rr, pl.ds(0, 16)] = (
                                jnp.full((16,), 1, jnp.int32) * pid_l)
                        pltpu.async_copy(dval16r, mapm.at[didx16], sems).wait()
                        return carry4
                    lax.fori_loop(0, wr, put, 0)
                    return carry3
                lax.fori_loop(0, need, fix, 0)
            return carry2
        lax.fori_loop(0, VPG * jnp.minimum(t, 1), scan, 0)
        return carry
    lax.fori_loop(0, NG, _p2, 0)

    def _p3(p, carry):
        row0 = base + p * PG
        pltpu.sync_copy(mapm.at[pl.ds(row0, PG)], mp2)
        for k in range(PG // 16):
            k16 = k * 16
            m = mp2[k16, pl.ds(0, 16)]
            for l in range(1, 16):
                m = jnp.where(lane == l, mp2[k16 + l, pl.ds(0, 16)], m)
            spread = P + ((row0 + k16 + lane) & (ZPAD - 1))
            gidx[pl.ds(k16, 16)] = jnp.where(m < 0, spread, m)
        pltpu.async_copy(feat.at[gidx], prow, semg).wait()
        pltpu.sync_copy(prow, table.at[pl.ds(row0, PG)])
        return carry
    lax.fori_loop(0, NPG, _p3, 0)


_sc_scatter = functools.partial(
    pl.kernel,
    out_type=(jax.ShapeDtypeStruct((N, C), jnp.float32),
              jax.ShapeDtypeStruct((N + NW * TPAD + 64, 16), jnp.int32)),
    mesh=plsc.VectorSubcoreMesh(core_axis_name="c", subcore_axis_name="s"),
    compiler_params=pltpu.CompilerParams(use_tc_tiling_on_sc=False),
    scratch_types=[
        pltpu.VMEM((250, 16), jnp.int32),
        pltpu.VMEM((GROUP,), jnp.int32),
        pltpu.VMEM((GROUP,), jnp.int32),
        pltpu.VMEM((GROUP,), jnp.int32),
        pltpu.VMEM((GROUP,), jnp.int32),
        pltpu.VMEM((GROUP, 16), jnp.int32),
        pltpu.VMEM((GROUP, 16), jnp.int32),
        pltpu.VMEM((16,), jnp.int32),
        pltpu.VMEM((16,), jnp.int32),
        pltpu.VMEM((16, 16), jnp.int32),
        pltpu.VMEM((16, 16), jnp.int32),
        pltpu.VMEM((PG, 16), jnp.int32),
        pltpu.VMEM((PG,), jnp.int32),
        pltpu.VMEM((PG, C), jnp.float32),
        pltpu.SemaphoreType.DMA,
        pltpu.SemaphoreType.DMA,
        pltpu.SemaphoreType.DMA,
    ],
)(_sc_body)


def _t_body(in_ref, out_ref):
    out_ref[...] = jnp.transpose(in_ref[...], (0, 3, 1, 2))


XB = 16
_transpose = pl.pallas_call(
    _t_body,
    grid=(B, H // XB),
    in_specs=[pl.BlockSpec((1, XB, W, C), lambda ib, ix: (ib, ix, 0, 0))],
    out_specs=pl.BlockSpec((1, C, XB, W), lambda ib, ix: (ib, 0, ix, 0)),
    out_shape=jax.ShapeDtypeStruct((B, C, H, W), jnp.float32),
)


def kernel(pillar_features, coors, batch_size):
    ci = coors.astype(jnp.int32)
    bq = ci[:, 0]
    xq = ci[:, 1]
    yq = ci[:, 2]
    feat_ext = jnp.concatenate(
        [pillar_features.astype(jnp.float32),
         jnp.zeros((ZPAD, C), jnp.float32)], axis=0)
    pidrows = jnp.broadcast_to(
        jnp.arange(P, dtype=jnp.int32)[:, None], (P, 16))
    table, _ = _sc_scatter(feat_ext, bq, xq, yq, pidrows)
    return _transpose(table.reshape(B, H, W, C))
